# baseline (speedup 1.0000x reference)
import functools
import math

import jax
import jax.numpy as jnp
import numpy as np
from jax import lax
from jax.experimental import pallas as pl
from jax.experimental.pallas import tpu as pltpu
from jax.experimental.pallas import tpu_sc as plsc

_HIDDEN = 768
_NV = _HIDDEN // 16
_NC, _NS = 2, 16
_NW = _NC * _NS
_CH = 64


def _pe_row(seq_len: int, hidden: int) -> np.ndarray:
    norm = np.exp(np.arange(0, hidden, 2, dtype=np.float64)
                  * (-(math.log(10000.0) / hidden)))
    row = np.zeros((hidden,), dtype=np.float64)
    row[0::2] = np.sin(seq_len * norm)
    row[1::2] = np.cos(seq_len * norm)
    return row.astype(np.float32)


def _lanes_sum(x):
    for k in (1, 2, 4, 8):
        perm = lax.iota(jnp.int32, 16) ^ k
        x = x + jnp.take_along_axis(x, perm, axis=0, mode="promise_in_bounds")
    return x


def _ln_rows(buf, seq_v, bias_v, gamma_v, beta_v, row0):
    def row_body(r, carry):
        sq = seq_v[pl.ds(row0 + r, 16)][0]
        bb = sq * _HIDDEN
        acc = [jnp.zeros((16,), jnp.float32) for _ in range(4)]
        acc2 = [jnp.zeros((16,), jnp.float32) for _ in range(4)]
        for j in range(_NV):
            v = buf[r, pl.ds(j * 16, 16)] + bias_v[pl.ds(bb + j * 16, 16)]
            buf[r, pl.ds(j * 16, 16)] = v
            acc[j % 4] = acc[j % 4] + v
            acc2[j % 4] = acc2[j % 4] + v * v
        vsum = _lanes_sum((acc[0] + acc[1]) + (acc[2] + acc[3]))
        vsq = _lanes_sum((acc2[0] + acc2[1]) + (acc2[2] + acc2[3]))
        vmean = vsum * (1.0 / _HIDDEN)
        vvar = vsq * (1.0 / _HIDDEN) - vmean * vmean + 1e-12
        iv = jnp.full((16,), 0x5F3759DF, jnp.int32) - lax.shift_right_logical(
            plsc.bitcast(vvar, jnp.int32), 1)
        y = plsc.bitcast(iv, jnp.float32)
        for _ in range(3):
            y = y * (1.5 - 0.5 * vvar * y * y)
        vmy = vmean * y
        for j in range(_NV):
            t = buf[r, pl.ds(j * 16, 16)] * y - vmy
            buf[r, pl.ds(j * 16, 16)] = (
                t * gamma_v[pl.ds(j * 16, 16)] + beta_v[pl.ds(j * 16, 16)])
        return carry

    lax.fori_loop(0, _CH, row_body, jnp.int32(0))


def _sc_fused(table, idx3, seq2, bias2, gamma, beta):
    n_ch = idx3.shape[1]
    b_per_w = n_ch * _CH
    n = _NW * b_per_w
    mesh = plsc.VectorSubcoreMesh(core_axis_name="c", subcore_axis_name="s")

    @functools.partial(
        pl.kernel,
        mesh=mesh,
        compiler_params=pltpu.CompilerParams(needs_layout_passes=False),
        out_type=jax.ShapeDtypeStruct((n, _HIDDEN), jnp.float32),
        scratch_types=[
            pltpu.VMEM((n_ch, _CH), jnp.int32),
            pltpu.VMEM((b_per_w + 16,), jnp.int32),
            pltpu.VMEM((2 * _HIDDEN,), jnp.float32),
            pltpu.VMEM((_HIDDEN,), jnp.float32),
            pltpu.VMEM((_HIDDEN,), jnp.float32),
            pltpu.VMEM((_CH, _HIDDEN), jnp.float32),
            pltpu.VMEM((_CH, _HIDDEN), jnp.float32),
            pltpu.SemaphoreType.DMA,
            pltpu.SemaphoreType.DMA,
            pltpu.SemaphoreType.DMA,
            pltpu.SemaphoreType.DMA,
        ],
    )
    def k(table_hbm, idx_hbm, seq_hbm, bias_hbm, gamma_hbm, beta_hbm, out_hbm,
          idx_v, seq_v, bias_v, gamma_v, beta_v, buf0, buf1,
          gsem0, gsem1, wsem0, wsem1):
        wid = lax.axis_index("s") * _NC + lax.axis_index("c")
        base = wid * b_per_w
        pltpu.sync_copy(idx_hbm.at[wid], idx_v)
        pltpu.sync_copy(seq_hbm.at[wid], seq_v)
        pltpu.sync_copy(bias_hbm, bias_v)
        pltpu.sync_copy(gamma_hbm, gamma_v)
        pltpu.sync_copy(beta_hbm, beta_v)

        bufs = (buf0, buf1)
        gsems = (gsem0, gsem1)
        wsems = (wsem0, wsem1)
        gcp = [pltpu.async_copy(table_hbm.at[idx_v.at[i]], bufs[i], gsems[i])
               for i in range(min(2, n_ch))]
        wcp = []
        for i in range(n_ch):
            buf = bufs[i % 2]
            gcp[i].wait()
            _ln_rows(buf, seq_v, bias_v, gamma_v, beta_v, i * _CH)
            wcp.append(pltpu.async_copy(
                buf, out_hbm.at[pl.ds(base + i * _CH, _CH)], wsems[i % 2]))
            if i >= 1 and i + 1 < n_ch:
                wcp[i - 1].wait()
                gcp.append(pltpu.async_copy(
                    table_hbm.at[idx_v.at[i + 1]], bufs[(i + 1) % 2],
                    gsems[(i + 1) % 2]))
        wcp[-2].wait()
        wcp[-1].wait()

    return k(table, idx3, seq2, bias2, gamma, beta)


def kernel(inputIDs, sequenceIDs, word_table, seq_table, gamma, beta):
    b, l = inputIDs.shape
    n = b * l
    n_ch = n // (_NW * _CH)
    idx3 = inputIDs.reshape(_NW, n_ch, _CH).astype(jnp.int32)
    seq2 = sequenceIDs.reshape(_NW, n_ch * _CH).astype(jnp.int32)
    seq2 = jnp.concatenate(
        [seq2, jnp.zeros((_NW, 16), jnp.int32)], axis=1)

    pe = jnp.asarray(_pe_row(l, _HIDDEN))
    bias2 = (seq_table + pe[None, :]).reshape(2 * _HIDDEN)

    out = _sc_fused(word_table, idx3, seq2, bias2, gamma, beta)
    return out.reshape(b, l, _HIDDEN)


# speedup vs baseline: 2.6833x; 2.6833x over previous
import functools
import math

import jax
import jax.numpy as jnp
import numpy as np
from jax import lax
from jax.experimental import pallas as pl
from jax.experimental.pallas import tpu as pltpu
from jax.experimental.pallas import tpu_sc as plsc

_HIDDEN = 768
_NC, _NS = 2, 16
_NW = _NC * _NS
_K = 4
_BR = 512


def _pe_row(seq_len: int, hidden: int) -> np.ndarray:
    norm = np.exp(np.arange(0, hidden, 2, dtype=np.float64)
                  * (-(math.log(10000.0) / hidden)))
    row = np.zeros((hidden,), dtype=np.float64)
    row[0::2] = np.sin(seq_len * norm)
    row[1::2] = np.cos(seq_len * norm)
    return row.astype(np.float32)


def _sc_gather_chunk(table, idx2):
    ch = idx2.shape[1]
    n = _NW * ch
    mesh = plsc.VectorSubcoreMesh(core_axis_name="c", subcore_axis_name="s")

    @functools.partial(
        pl.kernel,
        mesh=mesh,
        out_type=jax.ShapeDtypeStruct((n, _HIDDEN), jnp.float32),
        scratch_types=[
            pltpu.VMEM((ch,), jnp.int32),
            pltpu.VMEM((ch, _HIDDEN), jnp.float32),
            pltpu.SemaphoreType.DMA,
        ],
    )
    def k(table_hbm, idx_hbm, out_hbm, idx_v, buf, sem):
        wid = lax.axis_index("s") * _NC + lax.axis_index("c")
        pltpu.sync_copy(idx_hbm.at[wid], idx_v)
        pltpu.async_copy(table_hbm.at[idx_v], buf, sem).wait()
        pltpu.sync_copy(buf, out_hbm.at[pl.ds(wid * ch, ch)])

    return k(table, idx2)


def _tc_ln_chunk(rows, seq_i, bias0, dbias, gamma, beta, c, out_prev):
    nc = rows.shape[0]
    n = nc * _K
    grid = (nc // _BR,)

    def body(rows_ref, seq_ref, b0_ref, db_ref, g_ref, be_ref, *rest):
        out_ref = rest[-1]
        x = rows_ref[...]
        s = seq_ref[...].astype(jnp.float32)
        x = x + b0_ref[...] + s * db_ref[...]
        mean = jnp.mean(x, axis=-1, keepdims=True)
        xc = x - mean
        var = jnp.mean(xc * xc, axis=-1, keepdims=True)
        rstd = lax.rsqrt(var + 1e-12)
        out_ref[...] = g_ref[...] * (xc * rstd) + be_ref[...]

    base = c * (nc // _BR)
    in_specs = [
        pl.BlockSpec((_BR, _HIDDEN), lambda i: (i, 0)),
        pl.BlockSpec((_BR, 1), lambda i: (i, 0)),
        pl.BlockSpec((1, _HIDDEN), lambda i: (0, 0)),
        pl.BlockSpec((1, _HIDDEN), lambda i: (0, 0)),
        pl.BlockSpec((1, _HIDDEN), lambda i: (0, 0)),
        pl.BlockSpec((1, _HIDDEN), lambda i: (0, 0)),
    ]
    args = [rows, seq_i, bias0, dbias, gamma, beta]
    aliases = {}
    if out_prev is not None:
        in_specs.append(pl.BlockSpec(memory_space=pl.ANY))
        args.append(out_prev)
        aliases = {6: 0}
    return pl.pallas_call(
        body,
        grid=grid,
        in_specs=in_specs,
        out_specs=pl.BlockSpec((_BR, _HIDDEN), lambda i: (base + i, 0)),
        out_shape=jax.ShapeDtypeStruct((n, _HIDDEN), jnp.float32),
        input_output_aliases=aliases,
    )(*args)


def kernel(inputIDs, sequenceIDs, word_table, seq_table, gamma, beta):
    b, l = inputIDs.shape
    n = b * l
    nc = n // _K
    ids3 = inputIDs.reshape(_K, _NW, nc // _NW).astype(jnp.int32)
    seq3 = sequenceIDs.reshape(_K, nc, 1).astype(jnp.int32)

    pe = jnp.asarray(_pe_row(l, _HIDDEN))
    bias0 = (seq_table[0] + pe).reshape(1, _HIDDEN)
    dbias = (seq_table[1] - seq_table[0]).reshape(1, _HIDDEN)
    gamma2 = gamma.reshape(1, _HIDDEN)
    beta2 = beta.reshape(1, _HIDDEN)

    gathered = [_sc_gather_chunk(word_table, ids3[c]) for c in range(_K)]
    out = None
    for c in range(_K):
        out = _tc_ln_chunk(gathered[c], seq3[c], bias0, dbias,
                           gamma2, beta2, c, out)
    return out.reshape(b, l, _HIDDEN)
